# baseline (speedup 1.0000x reference)
import functools

import jax
import jax.numpy as jnp
from jax import lax
from jax.experimental import pallas as pl
from jax.experimental.pallas import tpu as pltpu
from jax.experimental.pallas import tpu_sc as plsc

_F32 = jnp.float32
_NB = 8


def _shift_down(p, zrow):
    return jnp.concatenate([zrow, p[:-1]], axis=0)


def _shift_up(p, zrow):
    return jnp.concatenate([p[1:], zrow], axis=0)


def _ntdot(a, w):
    return lax.dot_general(a, w, (((1,), (1,)), ((), ())),
                           preferred_element_type=_F32)


def _enc_layer(pls, w0, w1, w2, w3, relu):
    U, C = pls[0][0].shape
    zrow = jnp.zeros((1, C), _F32)
    A = [[], [], [], []]
    for planes in pls:
        M = len(planes)
        for r in range(M // 2):
            A[0].append(planes[2 * r - 1] if r > 0
                        else _shift_down(planes[M - 1], zrow))
            A[1].append(planes[2 * r])
            A[2].append(planes[2 * r + 1])
            A[3].append(planes[2 * r + 2] if 2 * r + 2 < M
                        else _shift_up(planes[0], zrow))
    O = (_ntdot(jnp.concatenate(A[0], 0), w0)
         + _ntdot(jnp.concatenate(A[1], 0), w1)
         + _ntdot(jnp.concatenate(A[2], 0), w2)
         + _ntdot(jnp.concatenate(A[3], 0), w3))
    if relu:
        O = jnp.maximum(O, 0.0)
    out, off = [], 0
    for planes in pls:
        half = len(planes) // 2
        out.append([O[(off + i) * U:(off + i + 1) * U] for i in range(half)])
        off += half
    return out


def _enc_body(xv_ref, bigw_ref, wlo_ref, whi_ref, *refs):
    w2 = [refs[i][...] for i in range(4)]
    w3 = [refs[4 + i][...] for i in range(4)]
    w4 = [refs[8 + i][...] for i in range(4)]
    zt_ref = refs[12]
    V = jnp.concatenate([xv_ref[b] for b in range(_NB)], axis=0)
    big = jnp.dot(V, bigw_ref[...], preferred_element_type=_F32)
    zrow1 = jnp.zeros((1, 1), _F32)
    pls = []
    for b in range(_NB):
        v = V[b * 65:(b + 1) * 65]
        g = big[b * 65:(b + 1) * 65]
        corr0 = _shift_down(v[:, 15:16], zrow1) * wlo_ref[...]
        corr7 = _shift_up(v[:, 0:1], zrow1) * whi_ref[...]
        planes = [g[:, 32 * r:32 * (r + 1)] for r in range(8)]
        planes[0] = planes[0] + corr0
        planes[7] = planes[7] + corr7
        pls.append([jnp.maximum(p, 0.0) for p in planes])
    pls = _enc_layer(pls, *w2, True)
    pls = _enc_layer(pls, *w3, True)
    pls = _enc_layer(pls, *w4, False)
    for b in range(_NB):
        zt_ref[b] = pls[b][0].T


def _encoder(xv, bigw, wlo, whi, w2t, w3t, w4t):
    B = xv.shape[0]
    wspecs = [pl.BlockSpec(w.shape, lambda b: (0, 0))
              for w in (*w2t, *w3t, *w4t)]
    return pl.pallas_call(
        _enc_body,
        grid=(B // _NB,),
        in_specs=[
            pl.BlockSpec((_NB, 65, 16), lambda b: (b, 0, 0)),
            pl.BlockSpec((16, 256), lambda b: (0, 0)),
            pl.BlockSpec((1, 32), lambda b: (0, 0)),
            pl.BlockSpec((1, 32), lambda b: (0, 0)),
        ] + wspecs,
        out_specs=pl.BlockSpec((_NB, 128, 65), lambda b: (b, 0, 0)),
        out_shape=jax.ShapeDtypeStruct((B, 128, 65), _F32),
    )(xv, bigw, wlo, whi, *w2t, *w3t, *w4t)


def _dec_parts(pls):
    U, C = pls[0][0].shape
    zrow = jnp.zeros((1, C), _F32)
    P, Cc, N = [], [], []
    for planes in pls:
        M = len(planes)
        for r in range(M):
            P.append(planes[r - 1] if r > 0
                     else _shift_down(planes[M - 1], zrow))
            Cc.append(planes[r])
            N.append(planes[r + 1] if r < M - 1
                     else _shift_up(planes[0], zrow))
    return (jnp.concatenate(P, 0), jnp.concatenate(Cc, 0),
            jnp.concatenate(N, 0))


def _dec_layer(pls, w0, w1, w2, w3, relu):
    U = pls[0][0].shape[0]
    P, Cc, N = _dec_parts(pls)
    Ev = _ntdot(P, w0) + _ntdot(Cc, w2)
    Od = _ntdot(Cc, w1) + _ntdot(N, w3)
    if relu:
        Ev = jnp.maximum(Ev, 0.0)
        Od = jnp.maximum(Od, 0.0)
    out, off = [], 0
    for planes in pls:
        M = len(planes)
        cur = []
        for r in range(M):
            cur.append(Ev[(off + r) * U:(off + r + 1) * U])
            cur.append(Od[(off + r) * U:(off + r + 1) * U])
        out.append(cur)
        off += M
    return out


def _dec_body(zq_ref, zt_ref, *refs):
    w1 = [refs[i][...] for i in range(4)]
    w2 = [refs[4 + i][...] for i in range(4)]
    w3 = [refs[8 + i][...] for i in range(4)]
    w4_ref, o_ref, loss_ref = refs[12], refs[13], refs[14]

    d = zq_ref[...] - zt_ref[...]
    s = jnp.sum(d * d)
    k = pl.program_id(0)

    @pl.when(k == 0)
    def _():
        loss_ref[0, 0] = s

    @pl.when(k > 0)
    def _():
        loss_ref[0, 0] = loss_ref[0, 0] + s

    pls = [[zq_ref[b].T] for b in range(_NB)]
    pls = _dec_layer(pls, *w1, relu=True)
    pls = _dec_layer(pls, *w2, relu=True)
    pls = _dec_layer(pls, *w3, relu=True)
    xcat = []
    for planes in pls:
        U, C = planes[0].shape
        zrow = jnp.zeros((1, C), _F32)
        M = len(planes)
        for r in range(M):
            prev = (planes[r - 1] if r > 0
                    else _shift_down(planes[M - 1], zrow))
            nxt = (planes[r + 1] if r < M - 1
                   else _shift_up(planes[0], zrow))
            xcat.append(jnp.concatenate(
                [prev, planes[r], planes[r], nxt], axis=1))
    X4 = jnp.concatenate(xcat, 0)
    R = jnp.dot(X4, w4_ref[...], preferred_element_type=_F32)
    for b in range(_NB):
        o_ref[b] = jnp.concatenate(
            [R[(b * 8 + r) * 65:(b * 8 + r + 1) * 65] for r in range(8)],
            axis=1)


def _decoder(zq3, zT, w1t, w2t, w3t, w4):
    B = zq3.shape[0]
    wspecs = [pl.BlockSpec(w.shape, lambda b: (0, 0))
              for w in (*w1t, *w2t, *w3t)]
    return pl.pallas_call(
        _dec_body,
        grid=(B // _NB,),
        in_specs=[
            pl.BlockSpec((_NB, 128, 65), lambda b: (b, 0, 0)),
            pl.BlockSpec((_NB, 128, 65), lambda b: (b, 0, 0)),
        ] + wspecs + [pl.BlockSpec((128, 2), lambda b: (0, 0))],
        out_specs=[
            pl.BlockSpec((_NB, 65, 16), lambda b: (b, 0, 0)),
            pl.BlockSpec((1, 1), lambda b: (0, 0), memory_space=pltpu.SMEM),
        ],
        out_shape=[
            jax.ShapeDtypeStruct((B, 65, 16), _F32),
            jax.ShapeDtypeStruct((1, 1), _F32),
        ],
    )(zq3, zT, *w1t, *w2t, *w3t, w4)


def _dist_body(z_ref, e_ref, idx_ref, perp_ref, acc_ref, z2d_ref):
    k = pl.program_id(0)

    @pl.when(k == 0)
    def _():
        z2d_ref[...] = z_ref[...].reshape(z2d_ref.shape)

    KB = e_ref.shape[1]
    z = z2d_ref[:, pl.ds(k * KB, KB)]
    e = e_ref[...]
    part = -2.0 * lax.dot_general(
        z, e, (((1,), (1,)), ((), ())), preferred_element_type=_F32)
    part = part + jnp.sum(e * e, axis=1, keepdims=True).T

    @pl.when(k == 0)
    def _():
        acc_ref[...] = part

    @pl.when(k > 0)
    def _():
        acc_ref[...] = acc_ref[...] + part

    @pl.when(k == pl.num_programs(0) - 1)
    def _():
        acc = acc_ref[...]
        minval = jnp.min(acc, axis=1, keepdims=True)
        colids = lax.broadcasted_iota(jnp.int32, acc.shape, 1)
        big = jnp.int32(2 ** 30)
        idx = jnp.min(jnp.where(acc == minval, colids, big),
                      axis=1, keepdims=True)
        row = idx.astype(_F32).T
        idx_ref[...] = jnp.broadcast_to(row, idx_ref.shape).astype(jnp.int32)
        onehot = (colids == idx).astype(_F32)
        e_mean = jnp.sum(onehot, axis=0, keepdims=True) / acc.shape[0]
        perp = jnp.exp(-jnp.sum(e_mean * jnp.log(e_mean + 1e-10)))
        perp_ref[0, 0] = perp


def _vq_argmin(zT, E):
    B = zT.shape[0]
    K, D = E.shape
    KB = 1664
    grid = D // KB
    idx2d, perp = pl.pallas_call(
        _dist_body,
        grid=(grid,),
        in_specs=[
            pl.BlockSpec((B, 128, 65), lambda i: (0, 0, 0)),
            pl.BlockSpec((K, KB), lambda i: (0, i)),
        ],
        out_specs=[
            pl.BlockSpec((8, B), lambda i: (0, 0)),
            pl.BlockSpec((1, 1), lambda i: (0, 0), memory_space=pltpu.SMEM),
        ],
        out_shape=[
            jax.ShapeDtypeStruct((8, B), jnp.int32),
            jax.ShapeDtypeStruct((1, 1), _F32),
        ],
        scratch_shapes=[pltpu.VMEM((B, K), _F32), pltpu.VMEM((B, D), _F32)],
    )(zT, E)
    return idx2d, perp[0, 0]


def _sc_gather(E, idx2d):
    B = idx2d.shape[1]
    D = E.shape[1]
    rows_per_w = 8
    n_workers = B // rows_per_w
    mesh = plsc.VectorSubcoreMesh(core_axis_name="c", subcore_axis_name="s")

    @functools.partial(
        pl.kernel,
        mesh=mesh,
        out_type=jax.ShapeDtypeStruct((B, D), _F32),
        scratch_types=[
            pltpu.VMEM((rows_per_w,), jnp.int32),
            pltpu.VMEM((rows_per_w, D), _F32),
            pltpu.SemaphoreType.DMA,
        ],
    )
    def gather_kernel(idx_hbm, table_hbm, out_hbm, idx_v, rows_v, sem):
        wid = lax.axis_index("s") * 2 + lax.axis_index("c")

        @pl.when(wid < n_workers)
        def _():
            base = wid * rows_per_w
            pltpu.sync_copy(idx_hbm.at[0, pl.ds(base, rows_per_w)], idx_v)
            pltpu.async_copy(table_hbm.at[idx_v], rows_v, sem).wait()
            pltpu.sync_copy(rows_v, out_hbm.at[pl.ds(base, rows_per_w)])

    return gather_kernel(idx2d, E)


def _heads_body(zt_ref, wm_ref, bm_ref, wa1_ref, ba1_ref,
                wa2_ref, ba2_ref, wa3_ref, ba3_ref, mt_ref, adv_ref):
    zz = zt_ref[...]
    B = zz.shape[0]
    zm = zz[:, :64, :].reshape(B, 4160)
    za = zz[:, 64:, :].reshape(B, 4160)
    mt_ref[...] = (jnp.dot(zm, wm_ref[...],
                           preferred_element_type=_F32) + bm_ref[...])
    a = jnp.dot(za, wa1_ref[...], preferred_element_type=_F32)
    a = jnp.maximum(a + ba1_ref[...], 0.0)
    a = jnp.dot(a, wa2_ref[...], preferred_element_type=_F32)
    a = jnp.maximum(a + ba2_ref[...], 0.0)
    a = jnp.dot(a, wa3_ref[...], preferred_element_type=_F32)
    adv_ref[...] = a + ba3_ref[...]


def _heads(zT, wm, bm2, wa1, ba12, wa2, ba22, wa3, ba32):
    B = zT.shape[0]
    NC = wm.shape[1]
    full = lambda a: pl.BlockSpec(a.shape, lambda: tuple([0] * a.ndim))
    args = (zT, wm, bm2, wa1, ba12, wa2, ba22, wa3, ba32)
    return pl.pallas_call(
        _heads_body,
        in_specs=[full(a) for a in args],
        out_specs=[pl.BlockSpec((B, NC), lambda: (0, 0))] * 2,
        out_shape=[jax.ShapeDtypeStruct((B, NC), _F32)] * 2,
    )(*args)


def kernel(x, We1, We2, We3, We4, E, Wm, bm, Wa1, ba1, Wa2, ba2, Wa3, ba3,
           Wd1, Wd2, Wd3, Wd4):
    B, T = x.shape

    xv = x.reshape(B, 65, 16)
    w1cat = We1.transpose(2, 1, 0).reshape(4, 32)
    bigw = jnp.zeros((16, 8, 32), _F32)
    for r in range(8):
        for j in range(4):
            k = 2 * r - 1 + j
            if 0 <= k < 16:
                bigw = bigw.at[k, r].set(w1cat[j])
    bigw = bigw.reshape(16, 256)
    wlo = w1cat[0:1]
    whi = w1cat[3:4]

    taps = lambda w: tuple(w[:, :, j] for j in range(4))
    zT = _encoder(xv, bigw, wlo, whi, taps(We2), taps(We3), taps(We4))

    idx2d, perplexity = _vq_argmin(zT, E)
    zq_flat = _sc_gather(E, idx2d)

    r2 = lambda v: v.reshape(1, -1)
    multitask, adversary = _heads(
        zT, Wm, r2(bm), Wa1, r2(ba1), Wa2, r2(ba2), Wa3, r2(ba3))

    wt4 = Wd4.transpose(2, 1, 0)
    z32 = jnp.zeros((64, 1), _F32)
    w4 = jnp.concatenate([
        jnp.concatenate([jnp.concatenate([wt4[0], wt4[2]], 0), z32], 1),
        jnp.concatenate([z32, jnp.concatenate([wt4[1], wt4[3]], 0)], 1),
    ], axis=0)

    xh, loss = _decoder(zq_flat.reshape(B, 128, 65), zT,
                        taps(Wd1), taps(Wd2), taps(Wd3), w4)
    x_hat = xh.reshape(B, 1, 1040)
    embedding_loss = loss[0, 0] * (1.25 / (B * 8320))

    return (embedding_loss, x_hat, multitask, adversary, perplexity)


# speedup vs baseline: 1.1264x; 1.1264x over previous
import functools

import jax
import jax.numpy as jnp
from jax import lax
from jax.experimental import pallas as pl
from jax.experimental.pallas import tpu as pltpu
from jax.experimental.pallas import tpu_sc as plsc

_F32 = jnp.float32
_NB = 8


def _shift_down(p, zrow):
    return jnp.concatenate([zrow, p[:-1]], axis=0)


def _shift_up(p, zrow):
    return jnp.concatenate([p[1:], zrow], axis=0)


def _ntdot(a, w):
    return lax.dot_general(a, w, (((1,), (1,)), ((), ())),
                           preferred_element_type=_F32)


def _enc_layer(pls, wcat, relu):
    U, C = pls[0][0].shape
    zrow = jnp.zeros((1, C), _F32)
    blocks = []
    for planes in pls:
        M = len(planes)
        for r in range(M // 2):
            a0 = (planes[2 * r - 1] if r > 0
                  else _shift_down(planes[M - 1], zrow))
            a1 = planes[2 * r]
            a2 = planes[2 * r + 1]
            a3 = (planes[2 * r + 2] if 2 * r + 2 < M
                  else _shift_up(planes[0], zrow))
            blocks.append(jnp.concatenate([a0, a1, a2, a3], axis=1))
    X = jnp.concatenate(blocks, axis=0)
    O = jnp.dot(X, wcat, preferred_element_type=_F32)
    if relu:
        O = jnp.maximum(O, 0.0)
    out, off = [], 0
    for planes in pls:
        half = len(planes) // 2
        out.append([O[(off + i) * U:(off + i + 1) * U] for i in range(half)])
        off += half
    return out


def _enc_body(xv_ref, bigw_ref, wlo_ref, whi_ref, w2_ref, w3_ref, w4_ref,
              zt_ref):
    V = jnp.concatenate([xv_ref[b] for b in range(_NB)], axis=0)
    big = jnp.dot(V, bigw_ref[...], preferred_element_type=_F32)
    zrow1 = jnp.zeros((1, 1), _F32)
    pls = []
    for b in range(_NB):
        v = V[b * 65:(b + 1) * 65]
        g = big[b * 65:(b + 1) * 65]
        corr0 = _shift_down(v[:, 15:16], zrow1) * wlo_ref[...]
        corr7 = _shift_up(v[:, 0:1], zrow1) * whi_ref[...]
        planes = [g[:, 32 * r:32 * (r + 1)] for r in range(8)]
        planes[0] = planes[0] + corr0
        planes[7] = planes[7] + corr7
        pls.append([jnp.maximum(p, 0.0) for p in planes])
    pls = _enc_layer(pls, w2_ref[...], True)
    pls = _enc_layer(pls, w3_ref[...], True)
    pls = _enc_layer(pls, w4_ref[...], False)
    for b in range(_NB):
        zt_ref[b] = pls[b][0].T


def _encoder(xv, bigw, wlo, whi, w2, w3, w4):
    B = xv.shape[0]
    return pl.pallas_call(
        _enc_body,
        grid=(B // _NB,),
        in_specs=[
            pl.BlockSpec((_NB, 65, 16), lambda b: (b, 0, 0)),
            pl.BlockSpec((16, 256), lambda b: (0, 0)),
            pl.BlockSpec((1, 32), lambda b: (0, 0)),
            pl.BlockSpec((1, 32), lambda b: (0, 0)),
            pl.BlockSpec((128, 64), lambda b: (0, 0)),
            pl.BlockSpec((256, 128), lambda b: (0, 0)),
            pl.BlockSpec((512, 128), lambda b: (0, 0)),
        ],
        out_specs=pl.BlockSpec((_NB, 128, 65), lambda b: (b, 0, 0)),
        out_shape=jax.ShapeDtypeStruct((B, 128, 65), _F32),
    )(xv, bigw, wlo, whi, w2, w3, w4)


def _dec_shifted(planes):
    U, C = planes[0].shape
    M = len(planes)
    zrow = jnp.zeros((1, C), _F32)
    prev = [planes[r - 1] if r > 0 else _shift_down(planes[M - 1], zrow)
            for r in range(M)]
    nxt = [planes[r + 1] if r < M - 1 else _shift_up(planes[0], zrow)
           for r in range(M)]
    xe = [jnp.concatenate([prev[r], planes[r]], axis=1) for r in range(M)]
    xo = [jnp.concatenate([planes[r], nxt[r]], axis=1) for r in range(M)]
    return xe, xo


def _dec_layer(pls, we, wo, relu):
    U = pls[0][0].shape[0]
    xes, xos = [], []
    for planes in pls:
        xe, xo = _dec_shifted(planes)
        xes += xe
        xos += xo
    Ev = jnp.dot(jnp.concatenate(xes, 0), we, preferred_element_type=_F32)
    Od = jnp.dot(jnp.concatenate(xos, 0), wo, preferred_element_type=_F32)
    if relu:
        Ev = jnp.maximum(Ev, 0.0)
        Od = jnp.maximum(Od, 0.0)
    out, off = [], 0
    for planes in pls:
        M = len(planes)
        cur = []
        for r in range(M):
            cur.append(Ev[(off + r) * U:(off + r + 1) * U])
            cur.append(Od[(off + r) * U:(off + r + 1) * U])
        out.append(cur)
        off += M
    return out


def _dec_body(zq_ref, zt_ref, w1e_ref, w1o_ref, w2e_ref, w2o_ref,
              w3e_ref, w3o_ref, w4_ref, o_ref, loss_ref):
    d = zq_ref[...] - zt_ref[...]
    s = jnp.sum(d * d)
    k = pl.program_id(0)

    @pl.when(k == 0)
    def _():
        loss_ref[0, 0] = s

    @pl.when(k > 0)
    def _():
        loss_ref[0, 0] = loss_ref[0, 0] + s

    pls = [[zq_ref[b].T] for b in range(_NB)]
    pls = _dec_layer(pls, w1e_ref[...], w1o_ref[...], True)
    pls = _dec_layer(pls, w2e_ref[...], w2o_ref[...], True)
    pls = _dec_layer(pls, w3e_ref[...], w3o_ref[...], True)
    xcat = []
    for planes in pls:
        xe, xo = _dec_shifted(planes)
        xcat += [jnp.concatenate([xe[r], xo[r]], axis=1) for r in range(8)]
    X4 = jnp.concatenate(xcat, 0)
    R = jnp.dot(X4, w4_ref[...], preferred_element_type=_F32)
    for b in range(_NB):
        o_ref[b] = jnp.concatenate(
            [R[(b * 8 + r) * 65:(b * 8 + r + 1) * 65] for r in range(8)],
            axis=1)


def _decoder(zq3, zT, w1e, w1o, w2e, w2o, w3e, w3o, w4):
    B = zq3.shape[0]
    return pl.pallas_call(
        _dec_body,
        grid=(B // _NB,),
        in_specs=[
            pl.BlockSpec((_NB, 128, 65), lambda b: (b, 0, 0)),
            pl.BlockSpec((_NB, 128, 65), lambda b: (b, 0, 0)),
            pl.BlockSpec((256, 128), lambda b: (0, 0)),
            pl.BlockSpec((256, 128), lambda b: (0, 0)),
            pl.BlockSpec((256, 64), lambda b: (0, 0)),
            pl.BlockSpec((256, 64), lambda b: (0, 0)),
            pl.BlockSpec((128, 32), lambda b: (0, 0)),
            pl.BlockSpec((128, 32), lambda b: (0, 0)),
            pl.BlockSpec((128, 2), lambda b: (0, 0)),
        ],
        out_specs=[
            pl.BlockSpec((_NB, 65, 16), lambda b: (b, 0, 0)),
            pl.BlockSpec((1, 1), lambda b: (0, 0), memory_space=pltpu.SMEM),
        ],
        out_shape=[
            jax.ShapeDtypeStruct((B, 65, 16), _F32),
            jax.ShapeDtypeStruct((1, 1), _F32),
        ],
    )(zq3, zT, w1e, w1o, w2e, w2o, w3e, w3o, w4)


def _dist_body(z_ref, e_ref, idx_ref, perp_ref, acc_ref, z2d_ref):
    k = pl.program_id(0)

    @pl.when(k == 0)
    def _():
        z2d_ref[...] = z_ref[...].reshape(z2d_ref.shape)

    KB = e_ref.shape[1]
    z = z2d_ref[:, pl.ds(k * KB, KB)]
    e = e_ref[...]
    part = -2.0 * lax.dot_general(
        z, e, (((1,), (1,)), ((), ())), preferred_element_type=_F32)
    part = part + jnp.sum(e * e, axis=1, keepdims=True).T

    @pl.when(k == 0)
    def _():
        acc_ref[...] = part

    @pl.when(k > 0)
    def _():
        acc_ref[...] = acc_ref[...] + part

    @pl.when(k == pl.num_programs(0) - 1)
    def _():
        acc = acc_ref[...]
        minval = jnp.min(acc, axis=1, keepdims=True)
        colids = lax.broadcasted_iota(jnp.int32, acc.shape, 1)
        big = jnp.int32(2 ** 30)
        idx = jnp.min(jnp.where(acc == minval, colids, big),
                      axis=1, keepdims=True)
        idx_ref[...] = jnp.broadcast_to(idx, idx_ref.shape)
        onehot = (colids == idx).astype(_F32)
        e_mean = jnp.sum(onehot, axis=0, keepdims=True) / acc.shape[0]
        perp = jnp.exp(-jnp.sum(e_mean * jnp.log(e_mean + 1e-10)))
        perp_ref[0, 0] = perp


def _vq_argmin(zT, E):
    B = zT.shape[0]
    K, D = E.shape
    KB = 1664
    grid = D // KB
    idx2d, perp = pl.pallas_call(
        _dist_body,
        grid=(grid,),
        in_specs=[
            pl.BlockSpec((B, 128, 65), lambda i: (0, 0, 0)),
            pl.BlockSpec((K, KB), lambda i: (0, i)),
        ],
        out_specs=[
            pl.BlockSpec((B, 128), lambda i: (0, 0)),
            pl.BlockSpec((1, 1), lambda i: (0, 0), memory_space=pltpu.SMEM),
        ],
        out_shape=[
            jax.ShapeDtypeStruct((B, 128), jnp.int32),
            jax.ShapeDtypeStruct((1, 1), _F32),
        ],
        scratch_shapes=[pltpu.VMEM((B, K), _F32), pltpu.VMEM((B, D), _F32)],
    )(zT, E)
    return idx2d[:, 0], perp[0, 0]


def _sc_gather(E, idx):
    B = idx.shape[0]
    D = E.shape[1]
    rows_per_w = 8
    n_workers = B // rows_per_w
    mesh = plsc.VectorSubcoreMesh(core_axis_name="c", subcore_axis_name="s")

    @functools.partial(
        pl.kernel,
        mesh=mesh,
        out_type=jax.ShapeDtypeStruct((B, D), _F32),
        scratch_types=[
            pltpu.VMEM((rows_per_w,), jnp.int32),
            pltpu.VMEM((rows_per_w, D), _F32),
            pltpu.SemaphoreType.DMA,
        ],
    )
    def gather_kernel(idx_hbm, table_hbm, out_hbm, idx_v, rows_v, sem):
        wid = lax.axis_index("s") * 2 + lax.axis_index("c")

        @pl.when(wid < n_workers)
        def _():
            base = wid * rows_per_w
            pltpu.sync_copy(idx_hbm.at[pl.ds(base, rows_per_w)], idx_v)
            pltpu.async_copy(table_hbm.at[idx_v], rows_v, sem).wait()
            pltpu.sync_copy(rows_v, out_hbm.at[pl.ds(base, rows_per_w)])

    return gather_kernel(idx, E)


def _heads_body(zt_ref, wm_ref, bm_ref, wa1_ref, ba1_ref,
                wa2_ref, ba2_ref, wa3_ref, ba3_ref, mt_ref, adv_ref):
    zz = zt_ref[...]
    B = zz.shape[0]
    zm = zz[:, :64, :].reshape(B, 4160)
    za = zz[:, 64:, :].reshape(B, 4160)
    mt_ref[...] = (jnp.dot(zm, wm_ref[...],
                           preferred_element_type=_F32) + bm_ref[...])
    a = jnp.dot(za, wa1_ref[...], preferred_element_type=_F32)
    a = jnp.maximum(a + ba1_ref[...], 0.0)
    a = jnp.dot(a, wa2_ref[...], preferred_element_type=_F32)
    a = jnp.maximum(a + ba2_ref[...], 0.0)
    a = jnp.dot(a, wa3_ref[...], preferred_element_type=_F32)
    adv_ref[...] = a + ba3_ref[...]


def _heads(zT, wm, bm2, wa1, ba12, wa2, ba22, wa3, ba32):
    B = zT.shape[0]
    NC = wm.shape[1]
    full = lambda a: pl.BlockSpec(a.shape, lambda: tuple([0] * a.ndim))
    args = (zT, wm, bm2, wa1, ba12, wa2, ba22, wa3, ba32)
    return pl.pallas_call(
        _heads_body,
        in_specs=[full(a) for a in args],
        out_specs=[pl.BlockSpec((B, NC), lambda: (0, 0))] * 2,
        out_shape=[jax.ShapeDtypeStruct((B, NC), _F32)] * 2,
    )(*args)


def kernel(x, We1, We2, We3, We4, E, Wm, bm, Wa1, ba1, Wa2, ba2, Wa3, ba3,
           Wd1, Wd2, Wd3, Wd4):
    B, T = x.shape

    xv = x.reshape(B, 65, 16)
    w1cat = We1.transpose(2, 1, 0).reshape(4, 32)
    bigw = jnp.zeros((16, 8, 32), _F32)
    for r in range(8):
        for j in range(4):
            k = 2 * r - 1 + j
            if 0 <= k < 16:
                bigw = bigw.at[k, r].set(w1cat[j])
    bigw = bigw.reshape(16, 256)
    wlo = w1cat[0:1]
    whi = w1cat[3:4]

    enc_w = lambda w: w.transpose(2, 1, 0).reshape(-1, w.shape[0])
    zT = _encoder(xv, bigw, wlo, whi, enc_w(We2), enc_w(We3), enc_w(We4))

    idx, perplexity = _vq_argmin(zT, E)
    zq_flat = _sc_gather(E, idx)

    r2 = lambda v: v.reshape(1, -1)
    multitask, adversary = _heads(
        zT, Wm, r2(bm), Wa1, r2(ba1), Wa2, r2(ba2), Wa3, r2(ba3))

    def dec_w(w):
        wt = w.transpose(2, 1, 0)
        return (jnp.concatenate([wt[0], wt[2]], axis=0),
                jnp.concatenate([wt[1], wt[3]], axis=0))

    w1e, w1o = dec_w(Wd1)
    w2e, w2o = dec_w(Wd2)
    w3e, w3o = dec_w(Wd3)
    wt4 = Wd4.transpose(2, 1, 0)
    z32 = jnp.zeros((64, 1), _F32)
    w4 = jnp.concatenate([
        jnp.concatenate([jnp.concatenate([wt4[0], wt4[2]], 0), z32], 1),
        jnp.concatenate([z32, jnp.concatenate([wt4[1], wt4[3]], 0)], 1),
    ], axis=0)

    xh, loss = _decoder(zq_flat.reshape(B, 128, 65), zT,
                        w1e, w1o, w2e, w2o, w3e, w3o, w4)
    x_hat = xh.reshape(B, 1, 1040)
    embedding_loss = loss[0, 0] * (1.25 / (B * 8320))

    return (embedding_loss, x_hat, multitask, adversary, perplexity)


# speedup vs baseline: 1.1764x; 1.0444x over previous
import functools

import jax
import jax.numpy as jnp
from jax import lax
from jax.experimental import pallas as pl
from jax.experimental.pallas import tpu as pltpu
from jax.experimental.pallas import tpu_sc as plsc

_F32 = jnp.float32
_NB = 8


def _shift_down(p, zrow):
    return jnp.concatenate([zrow, p[:-1]], axis=0)


def _shift_up(p, zrow):
    return jnp.concatenate([p[1:], zrow], axis=0)


def _ntdot(a, w):
    return lax.dot_general(a, w, (((1,), (1,)), ((), ())),
                           preferred_element_type=_F32)


def _enc_layer(pls, wcat, relu):
    U, C = pls[0][0].shape
    zrow = jnp.zeros((1, C), _F32)
    blocks = []
    for planes in pls:
        M = len(planes)
        for r in range(M // 2):
            a0 = (planes[2 * r - 1] if r > 0
                  else _shift_down(planes[M - 1], zrow))
            a1 = planes[2 * r]
            a2 = planes[2 * r + 1]
            a3 = (planes[2 * r + 2] if 2 * r + 2 < M
                  else _shift_up(planes[0], zrow))
            blocks.append(jnp.concatenate([a0, a1, a2, a3], axis=1))
    X = jnp.concatenate(blocks, axis=0)
    O = jnp.dot(X, wcat, preferred_element_type=_F32)
    if relu:
        O = jnp.maximum(O, 0.0)
    out, off = [], 0
    for planes in pls:
        half = len(planes) // 2
        out.append([O[(off + i) * U:(off + i + 1) * U] for i in range(half)])
        off += half
    return out


def _enc_body(xv_ref, w1_ref, w2_ref, w3_ref, w4_ref, zt_ref):
    zrow1 = jnp.zeros((1, 1), _F32)
    blocks = []
    for b in range(_NB):
        v = xv_ref[b]
        for r in range(8):
            if r == 0:
                blk = jnp.concatenate(
                    [_shift_down(v[:, 15:16], zrow1), v[:, 0:3]], axis=1)
            elif r == 7:
                blk = jnp.concatenate(
                    [v[:, 13:16], _shift_up(v[:, 0:1], zrow1)], axis=1)
            else:
                blk = v[:, 2 * r - 1:2 * r + 3]
            blocks.append(blk)
    X1 = jnp.concatenate(blocks, axis=0)
    h = jnp.dot(X1, w1_ref[...], preferred_element_type=_F32)
    h = jnp.maximum(h, 0.0)
    pls = [[h[(b * 8 + r) * 65:(b * 8 + r + 1) * 65] for r in range(8)]
           for b in range(_NB)]
    pls = _enc_layer(pls, w2_ref[...], True)
    pls = _enc_layer(pls, w3_ref[...], True)
    pls = _enc_layer(pls, w4_ref[...], False)
    for b in range(_NB):
        zt_ref[b] = pls[b][0].T


def _encoder(xv, w1, w2, w3, w4):
    B = xv.shape[0]
    return pl.pallas_call(
        _enc_body,
        grid=(B // _NB,),
        in_specs=[
            pl.BlockSpec((_NB, 65, 16), lambda b: (b, 0, 0)),
            pl.BlockSpec((4, 32), lambda b: (0, 0)),
            pl.BlockSpec((128, 64), lambda b: (0, 0)),
            pl.BlockSpec((256, 128), lambda b: (0, 0)),
            pl.BlockSpec((512, 128), lambda b: (0, 0)),
        ],
        out_specs=pl.BlockSpec((_NB, 128, 65), lambda b: (b, 0, 0)),
        out_shape=jax.ShapeDtypeStruct((B, 128, 65), _F32),
    )(xv, w1, w2, w3, w4)


def _dec_shifted(planes):
    U, C = planes[0].shape
    M = len(planes)
    zrow = jnp.zeros((1, C), _F32)
    prev = [planes[r - 1] if r > 0 else _shift_down(planes[M - 1], zrow)
            for r in range(M)]
    nxt = [planes[r + 1] if r < M - 1 else _shift_up(planes[0], zrow)
           for r in range(M)]
    xe = [jnp.concatenate([prev[r], planes[r]], axis=1) for r in range(M)]
    xo = [jnp.concatenate([planes[r], nxt[r]], axis=1) for r in range(M)]
    return xe, xo


def _dec_layer(pls, we, wo, relu):
    U = pls[0][0].shape[0]
    xes, xos = [], []
    for planes in pls:
        xe, xo = _dec_shifted(planes)
        xes += xe
        xos += xo
    Ev = jnp.dot(jnp.concatenate(xes, 0), we, preferred_element_type=_F32)
    Od = jnp.dot(jnp.concatenate(xos, 0), wo, preferred_element_type=_F32)
    if relu:
        Ev = jnp.maximum(Ev, 0.0)
        Od = jnp.maximum(Od, 0.0)
    out, off = [], 0
    for planes in pls:
        M = len(planes)
        cur = []
        for r in range(M):
            cur.append(Ev[(off + r) * U:(off + r + 1) * U])
            cur.append(Od[(off + r) * U:(off + r + 1) * U])
        out.append(cur)
        off += M
    return out


def _dec_body(zq_ref, zt_ref, w1e_ref, w1o_ref, w2e_ref, w2o_ref,
              w3e_ref, w3o_ref, w4_ref, o_ref, loss_ref):
    d = zq_ref[...] - zt_ref[...]
    s = jnp.sum(d * d)
    k = pl.program_id(0)

    @pl.when(k == 0)
    def _():
        loss_ref[0, 0] = s

    @pl.when(k > 0)
    def _():
        loss_ref[0, 0] = loss_ref[0, 0] + s

    pls = [[zq_ref[b].T] for b in range(_NB)]
    pls = _dec_layer(pls, w1e_ref[...], w1o_ref[...], True)
    pls = _dec_layer(pls, w2e_ref[...], w2o_ref[...], True)
    pls = _dec_layer(pls, w3e_ref[...], w3o_ref[...], True)
    xcat = []
    for planes in pls:
        xe, xo = _dec_shifted(planes)
        xcat += [jnp.concatenate([xe[r], xo[r]], axis=1) for r in range(8)]
    X4 = jnp.concatenate(xcat, 0)
    R = jnp.dot(X4, w4_ref[...], preferred_element_type=_F32)
    for b in range(_NB):
        o_ref[b] = jnp.concatenate(
            [R[(b * 8 + r) * 65:(b * 8 + r + 1) * 65] for r in range(8)],
            axis=1)


def _decoder(zq3, zT, w1e, w1o, w2e, w2o, w3e, w3o, w4):
    B = zq3.shape[0]
    return pl.pallas_call(
        _dec_body,
        grid=(B // _NB,),
        in_specs=[
            pl.BlockSpec((_NB, 128, 65), lambda b: (b, 0, 0)),
            pl.BlockSpec((_NB, 128, 65), lambda b: (b, 0, 0)),
            pl.BlockSpec((256, 128), lambda b: (0, 0)),
            pl.BlockSpec((256, 128), lambda b: (0, 0)),
            pl.BlockSpec((256, 64), lambda b: (0, 0)),
            pl.BlockSpec((256, 64), lambda b: (0, 0)),
            pl.BlockSpec((128, 32), lambda b: (0, 0)),
            pl.BlockSpec((128, 32), lambda b: (0, 0)),
            pl.BlockSpec((128, 2), lambda b: (0, 0)),
        ],
        out_specs=[
            pl.BlockSpec((_NB, 65, 16), lambda b: (b, 0, 0)),
            pl.BlockSpec((1, 1), lambda b: (0, 0), memory_space=pltpu.SMEM),
        ],
        out_shape=[
            jax.ShapeDtypeStruct((B, 65, 16), _F32),
            jax.ShapeDtypeStruct((1, 1), _F32),
        ],
    )(zq3, zT, w1e, w1o, w2e, w2o, w3e, w3o, w4)


def _dist_body(z_ref, e_ref, idx_ref, perp_ref, acc_ref, z2d_ref):
    k = pl.program_id(0)

    @pl.when(k == 0)
    def _():
        z2d_ref[...] = z_ref[...].reshape(z2d_ref.shape)

    z = z2d_ref[...]
    e = e_ref[...]
    zz = jnp.sum(z * z, axis=1, keepdims=True)
    ee = jnp.sum(e * e, axis=1, keepdims=True).T
    dot = lax.dot_general(z, e, (((1,), (1,)), ((), ())),
                          preferred_element_type=_F32)
    NBK = e.shape[0]
    acc_ref[:, pl.ds(k * NBK, NBK)] = (zz + ee) - 2.0 * dot

    @pl.when(k == pl.num_programs(0) - 1)
    def _():
        acc = acc_ref[...]
        minval = jnp.min(acc, axis=1, keepdims=True)
        colids = lax.broadcasted_iota(jnp.int32, acc.shape, 1)
        big = jnp.int32(2 ** 30)
        idx = jnp.min(jnp.where(acc == minval, colids, big),
                      axis=1, keepdims=True)
        idx_ref[...] = jnp.broadcast_to(idx, idx_ref.shape)
        onehot = (colids == idx).astype(_F32)
        e_mean = jnp.sum(onehot, axis=0, keepdims=True) / acc.shape[0]
        perp = jnp.exp(-jnp.sum(e_mean * jnp.log(e_mean + 1e-10)))
        perp_ref[0, 0] = perp


def _vq_argmin(zT, E):
    B = zT.shape[0]
    K, D = E.shape
    NBK = 256
    grid = K // NBK
    idx2d, perp = pl.pallas_call(
        _dist_body,
        grid=(grid,),
        in_specs=[
            pl.BlockSpec((B, 128, 65), lambda i: (0, 0, 0)),
            pl.BlockSpec((NBK, D), lambda i: (i, 0)),
        ],
        out_specs=[
            pl.BlockSpec((B, 128), lambda i: (0, 0)),
            pl.BlockSpec((1, 1), lambda i: (0, 0), memory_space=pltpu.SMEM),
        ],
        out_shape=[
            jax.ShapeDtypeStruct((B, 128), jnp.int32),
            jax.ShapeDtypeStruct((1, 1), _F32),
        ],
        scratch_shapes=[pltpu.VMEM((B, K), _F32), pltpu.VMEM((B, D), _F32)],
    )(zT, E)
    return idx2d[:, 0], perp[0, 0]


def _sc_gather(E, idx):
    B = idx.shape[0]
    D = E.shape[1]
    rows_per_w = 8
    n_workers = B // rows_per_w
    mesh = plsc.VectorSubcoreMesh(core_axis_name="c", subcore_axis_name="s")

    @functools.partial(
        pl.kernel,
        mesh=mesh,
        out_type=jax.ShapeDtypeStruct((B, D), _F32),
        scratch_types=[
            pltpu.VMEM((rows_per_w,), jnp.int32),
            pltpu.VMEM((rows_per_w, D), _F32),
            pltpu.SemaphoreType.DMA,
        ],
    )
    def gather_kernel(idx_hbm, table_hbm, out_hbm, idx_v, rows_v, sem):
        wid = lax.axis_index("s") * 2 + lax.axis_index("c")

        @pl.when(wid < n_workers)
        def _():
            base = wid * rows_per_w
            pltpu.sync_copy(idx_hbm.at[pl.ds(base, rows_per_w)], idx_v)
            pltpu.async_copy(table_hbm.at[idx_v], rows_v, sem).wait()
            pltpu.sync_copy(rows_v, out_hbm.at[pl.ds(base, rows_per_w)])

    return gather_kernel(idx, E)


def _heads_body(zt_ref, wm_ref, bm_ref, wa1_ref, ba1_ref,
                wa2_ref, ba2_ref, wa3_ref, ba3_ref, mt_ref, adv_ref):
    zz = zt_ref[...]
    B = zz.shape[0]
    zm = zz[:, :64, :].reshape(B, 4160)
    za = zz[:, 64:, :].reshape(B, 4160)
    mt_ref[...] = (jnp.dot(zm, wm_ref[...],
                           preferred_element_type=_F32) + bm_ref[...])
    a = jnp.dot(za, wa1_ref[...], preferred_element_type=_F32)
    a = jnp.maximum(a + ba1_ref[...], 0.0)
    a = jnp.dot(a, wa2_ref[...], preferred_element_type=_F32)
    a = jnp.maximum(a + ba2_ref[...], 0.0)
    a = jnp.dot(a, wa3_ref[...], preferred_element_type=_F32)
    adv_ref[...] = a + ba3_ref[...]


def _heads(zT, wm, bm2, wa1, ba12, wa2, ba22, wa3, ba32):
    B = zT.shape[0]
    NC = wm.shape[1]
    full = lambda a: pl.BlockSpec(a.shape, lambda: tuple([0] * a.ndim))
    args = (zT, wm, bm2, wa1, ba12, wa2, ba22, wa3, ba32)
    return pl.pallas_call(
        _heads_body,
        in_specs=[full(a) for a in args],
        out_specs=[pl.BlockSpec((B, NC), lambda: (0, 0))] * 2,
        out_shape=[jax.ShapeDtypeStruct((B, NC), _F32)] * 2,
    )(*args)


def kernel(x, We1, We2, We3, We4, E, Wm, bm, Wa1, ba1, Wa2, ba2, Wa3, ba3,
           Wd1, Wd2, Wd3, Wd4):
    B, T = x.shape

    xv = x.reshape(B, 65, 16)
    enc_w = lambda w: w.transpose(2, 1, 0).reshape(-1, w.shape[0])
    zT = _encoder(xv, enc_w(We1), enc_w(We2), enc_w(We3), enc_w(We4))

    idx, perplexity = _vq_argmin(zT, E)
    zq_flat = _sc_gather(E, idx)

    r2 = lambda v: v.reshape(1, -1)
    multitask, adversary = _heads(
        zT, Wm, r2(bm), Wa1, r2(ba1), Wa2, r2(ba2), Wa3, r2(ba3))

    def dec_w(w):
        wt = w.transpose(2, 1, 0)
        return (jnp.concatenate([wt[0], wt[2]], axis=0),
                jnp.concatenate([wt[1], wt[3]], axis=0))

    w1e, w1o = dec_w(Wd1)
    w2e, w2o = dec_w(Wd2)
    w3e, w3o = dec_w(Wd3)
    wt4 = Wd4.transpose(2, 1, 0)
    z32 = jnp.zeros((64, 1), _F32)
    w4 = jnp.concatenate([
        jnp.concatenate([jnp.concatenate([wt4[0], wt4[2]], 0), z32], 1),
        jnp.concatenate([z32, jnp.concatenate([wt4[1], wt4[3]], 0)], 1),
    ], axis=0)

    xh, loss = _decoder(zq_flat.reshape(B, 128, 65), zT,
                        w1e, w1o, w2e, w2o, w3e, w3o, w4)
    x_hat = xh.reshape(B, 1, 1040)
    embedding_loss = loss[0, 0] * (1.25 / (B * 8320))

    return (embedding_loss, x_hat, multitask, adversary, perplexity)


# speedup vs baseline: 1.1779x; 1.0013x over previous
import functools

import jax
import jax.numpy as jnp
from jax import lax
from jax.experimental import pallas as pl
from jax.experimental.pallas import tpu as pltpu
from jax.experimental.pallas import tpu_sc as plsc

_F32 = jnp.float32
_NB = 8


def _shift_down(p, zrow):
    return jnp.concatenate([zrow, p[:-1]], axis=0)


def _shift_up(p, zrow):
    return jnp.concatenate([p[1:], zrow], axis=0)


def _ntdot(a, w):
    return lax.dot_general(a, w, (((1,), (1,)), ((), ())),
                           preferred_element_type=_F32)


def _enc_layer(pls, wcat, relu):
    U, C = pls[0][0].shape
    zrow = jnp.zeros((1, C), _F32)
    blocks = []
    for planes in pls:
        M = len(planes)
        for r in range(M // 2):
            a0 = (planes[2 * r - 1] if r > 0
                  else _shift_down(planes[M - 1], zrow))
            a1 = planes[2 * r]
            a2 = planes[2 * r + 1]
            a3 = (planes[2 * r + 2] if 2 * r + 2 < M
                  else _shift_up(planes[0], zrow))
            blocks.append(jnp.concatenate([a0, a1, a2, a3], axis=1))
    X = jnp.concatenate(blocks, axis=0)
    O = jnp.dot(X, wcat, preferred_element_type=_F32)
    if relu:
        O = jnp.maximum(O, 0.0)
    out, off = [], 0
    for planes in pls:
        half = len(planes) // 2
        out.append([O[(off + i) * U:(off + i + 1) * U] for i in range(half)])
        off += half
    return out


def _enc_body(xv_ref, w1_ref, w2_ref, w3_ref, w4_ref, zt_ref):
    zrow1 = jnp.zeros((1, 1), _F32)
    blocks = []
    for b in range(_NB):
        v = xv_ref[b]
        for r in range(8):
            if r == 0:
                blk = jnp.concatenate(
                    [_shift_down(v[:, 15:16], zrow1), v[:, 0:3]], axis=1)
            elif r == 7:
                blk = jnp.concatenate(
                    [v[:, 13:16], _shift_up(v[:, 0:1], zrow1)], axis=1)
            else:
                blk = v[:, 2 * r - 1:2 * r + 3]
            blocks.append(blk)
    X1 = jnp.concatenate(blocks, axis=0)
    h = jnp.dot(X1, w1_ref[...], preferred_element_type=_F32)
    h = jnp.maximum(h, 0.0)
    pls = [[h[(b * 8 + r) * 65:(b * 8 + r + 1) * 65] for r in range(8)]
           for b in range(_NB)]
    pls = _enc_layer(pls, w2_ref[...], True)
    pls = _enc_layer(pls, w3_ref[...], True)
    pls = _enc_layer(pls, w4_ref[...], False)
    for b in range(_NB):
        zt_ref[b] = pls[b][0].T


def _encoder(xv, w1, w2, w3, w4):
    B = xv.shape[0]
    return pl.pallas_call(
        _enc_body,
        grid=(B // _NB,),
        in_specs=[
            pl.BlockSpec((_NB, 65, 16), lambda b: (b, 0, 0)),
            pl.BlockSpec((4, 32), lambda b: (0, 0)),
            pl.BlockSpec((128, 64), lambda b: (0, 0)),
            pl.BlockSpec((256, 128), lambda b: (0, 0)),
            pl.BlockSpec((512, 128), lambda b: (0, 0)),
        ],
        out_specs=pl.BlockSpec((_NB, 128, 65), lambda b: (b, 0, 0)),
        out_shape=jax.ShapeDtypeStruct((B, 128, 65), _F32),
    )(xv, w1, w2, w3, w4)


def _dec_shifted(planes):
    U, C = planes[0].shape
    M = len(planes)
    zrow = jnp.zeros((1, C), _F32)
    prev = [planes[r - 1] if r > 0 else _shift_down(planes[M - 1], zrow)
            for r in range(M)]
    nxt = [planes[r + 1] if r < M - 1 else _shift_up(planes[0], zrow)
           for r in range(M)]
    xe = [jnp.concatenate([prev[r], planes[r]], axis=1) for r in range(M)]
    xo = [jnp.concatenate([planes[r], nxt[r]], axis=1) for r in range(M)]
    return xe, xo


def _dec_layer(pls, we, wo, relu):
    U = pls[0][0].shape[0]
    xes, xos = [], []
    for planes in pls:
        xe, xo = _dec_shifted(planes)
        xes += xe
        xos += xo
    Ev = jnp.dot(jnp.concatenate(xes, 0), we, preferred_element_type=_F32)
    Od = jnp.dot(jnp.concatenate(xos, 0), wo, preferred_element_type=_F32)
    if relu:
        Ev = jnp.maximum(Ev, 0.0)
        Od = jnp.maximum(Od, 0.0)
    out, off = [], 0
    for planes in pls:
        M = len(planes)
        cur = []
        for r in range(M):
            cur.append(Ev[(off + r) * U:(off + r + 1) * U])
            cur.append(Od[(off + r) * U:(off + r + 1) * U])
        out.append(cur)
        off += M
    return out


def _dec_body(zq_ref, zt_ref, w1e_ref, w1o_ref, w2e_ref, w2o_ref,
              w3e_ref, w3o_ref, w4_ref, o_ref, loss_ref):
    d = zq_ref[...] - zt_ref[...]
    s = jnp.sum(d * d)
    k = pl.program_id(0)

    @pl.when(k == 0)
    def _():
        loss_ref[0, 0] = s

    @pl.when(k > 0)
    def _():
        loss_ref[0, 0] = loss_ref[0, 0] + s

    pls = [[zq_ref[b].T] for b in range(_NB)]
    pls = _dec_layer(pls, w1e_ref[...], w1o_ref[...], True)
    pls = _dec_layer(pls, w2e_ref[...], w2o_ref[...], True)
    pls = _dec_layer(pls, w3e_ref[...], w3o_ref[...], True)
    xcat = []
    for planes in pls:
        xe, xo = _dec_shifted(planes)
        xcat += [jnp.concatenate([xe[r], xo[r]], axis=1) for r in range(8)]
    X4 = jnp.concatenate(xcat, 0)
    R = jnp.dot(X4, w4_ref[...], preferred_element_type=_F32)
    for b in range(_NB):
        o_ref[b] = jnp.concatenate(
            [R[(b * 8 + r) * 65:(b * 8 + r + 1) * 65] for r in range(8)],
            axis=1)


def _decoder(zq3, zT, w1e, w1o, w2e, w2o, w3e, w3o, w4):
    B = zq3.shape[0]
    return pl.pallas_call(
        _dec_body,
        grid=(B // _NB,),
        in_specs=[
            pl.BlockSpec((_NB, 128, 65), lambda b: (b, 0, 0)),
            pl.BlockSpec((_NB, 128, 65), lambda b: (b, 0, 0)),
            pl.BlockSpec((256, 128), lambda b: (0, 0)),
            pl.BlockSpec((256, 128), lambda b: (0, 0)),
            pl.BlockSpec((256, 64), lambda b: (0, 0)),
            pl.BlockSpec((256, 64), lambda b: (0, 0)),
            pl.BlockSpec((128, 32), lambda b: (0, 0)),
            pl.BlockSpec((128, 32), lambda b: (0, 0)),
            pl.BlockSpec((128, 2), lambda b: (0, 0)),
        ],
        out_specs=[
            pl.BlockSpec((_NB, 65, 16), lambda b: (b, 0, 0)),
            pl.BlockSpec((1, 1), lambda b: (0, 0), memory_space=pltpu.SMEM),
        ],
        out_shape=[
            jax.ShapeDtypeStruct((B, 65, 16), _F32),
            jax.ShapeDtypeStruct((1, 1), _F32),
        ],
    )(zq3, zT, w1e, w1o, w2e, w2o, w3e, w3o, w4)


def _dist_body(z_ref, e_ref, idx_ref, perp_ref, acc_ref, z2d_ref):
    k = pl.program_id(0)

    @pl.when(k == 0)
    def _():
        z2d_ref[...] = z_ref[...].reshape(z2d_ref.shape)

    z = z2d_ref[...]
    e = e_ref[...]
    zz = jnp.sum(z * z, axis=1, keepdims=True)
    ee = jnp.sum(e * e, axis=1, keepdims=True).T
    dot = lax.dot_general(z, e, (((1,), (1,)), ((), ())),
                          preferred_element_type=_F32)
    NBK = e.shape[0]
    acc_ref[:, pl.ds(k * NBK, NBK)] = (zz + ee) - 2.0 * dot

    @pl.when(k == pl.num_programs(0) - 1)
    def _():
        acc = acc_ref[...]
        minval = jnp.min(acc, axis=1, keepdims=True)
        colids = lax.broadcasted_iota(jnp.int32, acc.shape, 1)
        big = jnp.int32(2 ** 30)
        idx = jnp.min(jnp.where(acc == minval, colids, big),
                      axis=1, keepdims=True)
        idx_ref[...] = jnp.broadcast_to(idx, idx_ref.shape)
        onehot = (colids == idx).astype(_F32)
        e_mean = jnp.sum(onehot, axis=0, keepdims=True) / acc.shape[0]
        perp = jnp.exp(-jnp.sum(e_mean * jnp.log(e_mean + 1e-10)))
        perp_ref[0, 0] = perp


def _vq_argmin(zT, E):
    B = zT.shape[0]
    K, D = E.shape
    NBK = 256
    grid = K // NBK
    idx2d, perp = pl.pallas_call(
        _dist_body,
        grid=(grid,),
        in_specs=[
            pl.BlockSpec((B, 128, 65), lambda i: (0, 0, 0)),
            pl.BlockSpec((NBK, D), lambda i: (i, 0)),
        ],
        out_specs=[
            pl.BlockSpec((B, 128), lambda i: (0, 0)),
            pl.BlockSpec((1, 1), lambda i: (0, 0), memory_space=pltpu.SMEM),
        ],
        out_shape=[
            jax.ShapeDtypeStruct((B, 128), jnp.int32),
            jax.ShapeDtypeStruct((1, 1), _F32),
        ],
        scratch_shapes=[pltpu.VMEM((B, K), _F32), pltpu.VMEM((B, D), _F32)],
    )(zT, E)
    return idx2d[:, 0], perp[0, 0]


def _sc_gather(E, idx):
    B = idx.shape[0]
    D = E.shape[1]
    rows_per_w = 16
    col_splits = 5
    DW = D // col_splits
    mesh = plsc.VectorSubcoreMesh(core_axis_name="c", subcore_axis_name="s")

    @functools.partial(
        pl.kernel,
        mesh=mesh,
        out_type=jax.ShapeDtypeStruct((B, D), _F32),
        scratch_types=[
            pltpu.VMEM((rows_per_w,), jnp.int32),
            pltpu.VMEM((rows_per_w, DW), _F32),
            pltpu.SemaphoreType.DMA,
        ],
    )
    def gather_kernel(idx_hbm, table_hbm, out_hbm, idx_v, rows_v, sem):
        wid = lax.axis_index("s") * 2 + lax.axis_index("c")
        n_workers = (B // rows_per_w) * col_splits

        @pl.when(wid < n_workers)
        def _():
            wr = wid // col_splits
            h = wid % col_splits
            base = wr * rows_per_w
            col = h * DW
            pltpu.sync_copy(idx_hbm.at[pl.ds(base, rows_per_w)], idx_v)
            pltpu.async_copy(table_hbm.at[idx_v, pl.ds(col, DW)], rows_v,
                             sem).wait()
            pltpu.sync_copy(
                rows_v, out_hbm.at[pl.ds(base, rows_per_w), pl.ds(col, DW)])

    return gather_kernel(idx, E)


def _heads_body(zt_ref, wm_ref, bm_ref, wa1_ref, ba1_ref,
                wa2_ref, ba2_ref, wa3_ref, ba3_ref, mt_ref, adv_ref):
    zz = zt_ref[...]
    B = zz.shape[0]
    zm = zz[:, :64, :].reshape(B, 4160)
    za = zz[:, 64:, :].reshape(B, 4160)
    mt_ref[...] = (jnp.dot(zm, wm_ref[...],
                           preferred_element_type=_F32) + bm_ref[...])
    a = jnp.dot(za, wa1_ref[...], preferred_element_type=_F32)
    a = jnp.maximum(a + ba1_ref[...], 0.0)
    a = jnp.dot(a, wa2_ref[...], preferred_element_type=_F32)
    a = jnp.maximum(a + ba2_ref[...], 0.0)
    a = jnp.dot(a, wa3_ref[...], preferred_element_type=_F32)
    adv_ref[...] = a + ba3_ref[...]


def _heads(zT, wm, bm2, wa1, ba12, wa2, ba22, wa3, ba32):
    B = zT.shape[0]
    NC = wm.shape[1]
    full = lambda a: pl.BlockSpec(a.shape, lambda: tuple([0] * a.ndim))
    args = (zT, wm, bm2, wa1, ba12, wa2, ba22, wa3, ba32)
    return pl.pallas_call(
        _heads_body,
        in_specs=[full(a) for a in args],
        out_specs=[pl.BlockSpec((B, NC), lambda: (0, 0))] * 2,
        out_shape=[jax.ShapeDtypeStruct((B, NC), _F32)] * 2,
    )(*args)


def kernel(x, We1, We2, We3, We4, E, Wm, bm, Wa1, ba1, Wa2, ba2, Wa3, ba3,
           Wd1, Wd2, Wd3, Wd4):
    B, T = x.shape

    xv = x.reshape(B, 65, 16)
    enc_w = lambda w: w.transpose(2, 1, 0).reshape(-1, w.shape[0])
    zT = _encoder(xv, enc_w(We1), enc_w(We2), enc_w(We3), enc_w(We4))

    idx, perplexity = _vq_argmin(zT, E)
    zq_flat = _sc_gather(E, idx)

    r2 = lambda v: v.reshape(1, -1)
    multitask, adversary = _heads(
        zT, Wm, r2(bm), Wa1, r2(ba1), Wa2, r2(ba2), Wa3, r2(ba3))

    def dec_w(w):
        wt = w.transpose(2, 1, 0)
        return (jnp.concatenate([wt[0], wt[2]], axis=0),
                jnp.concatenate([wt[1], wt[3]], axis=0))

    w1e, w1o = dec_w(Wd1)
    w2e, w2o = dec_w(Wd2)
    w3e, w3o = dec_w(Wd3)
    wt4 = Wd4.transpose(2, 1, 0)
    z32 = jnp.zeros((64, 1), _F32)
    w4 = jnp.concatenate([
        jnp.concatenate([jnp.concatenate([wt4[0], wt4[2]], 0), z32], 1),
        jnp.concatenate([z32, jnp.concatenate([wt4[1], wt4[3]], 0)], 1),
    ], axis=0)

    xh, loss = _decoder(zq_flat.reshape(B, 128, 65), zT,
                        w1e, w1o, w2e, w2o, w3e, w3o, w4)
    x_hat = xh.reshape(B, 1, 1040)
    embedding_loss = loss[0, 0] * (1.25 / (B * 8320))

    return (embedding_loss, x_hat, multitask, adversary, perplexity)
